# Initial kernel scaffold; baseline (speedup 1.0000x reference)
import jax
import jax.numpy as jnp
from jax.experimental import pallas as pl


def kernel(x, batch, W1, W2):
    raise NotImplementedError("write your pallas kernel here")


# speedup vs baseline: 3.7607x; 3.7607x over previous
import jax
import jax.numpy as jnp
from jax.experimental import pallas as pl
from jax.experimental.pallas import tpu as pltpu

_G = 512
_BR = 2560


def _pass_a(lo_ref, hi_ref, x_ref, b_ref, w1_ref, w2_ref, y_ref,
            sum_ref, max_ref):
    i = pl.program_id(0)

    @pl.when(i == 0)
    def _init():
        sum_ref[...] = jnp.zeros_like(sum_ref)
        max_ref[...] = jnp.full_like(max_ref, -jnp.inf)

    b = b_ref[0, :, :]
    x = x_ref[...]
    s_lo = lo_ref[i]
    s_hi = hi_ref[i]

    def body(s, carry):
        m = b == s
        mx = jnp.max(jnp.where(m, x, -jnp.inf), axis=0, keepdims=True)
        sm = jnp.sum(jnp.where(m, x, 0.0), axis=0, keepdims=True)
        max_ref[pl.ds(s, 1), :] = jnp.maximum(max_ref[pl.ds(s, 1), :], mx)
        sum_ref[pl.ds(s, 1), :] = sum_ref[pl.ds(s, 1), :] + sm
        return carry

    jax.lax.fori_loop(s_lo, s_hi + 1, body, 0)

    @pl.when(i == pl.num_programs(0) - 1)
    def _finish():
        mx = max_ref[...]
        mx = jnp.where(mx == -jnp.inf, 0.0, mx)
        sm = sum_ref[...]
        w1 = w1_ref[...]
        w2 = w2_ref[...]
        h1 = jnp.maximum(jnp.dot(mx, w1, preferred_element_type=jnp.float32), 0.0)
        o1 = jnp.dot(h1, w2, preferred_element_type=jnp.float32)
        h2 = jnp.maximum(jnp.dot(sm, w1, preferred_element_type=jnp.float32), 0.0)
        o2 = jnp.dot(h2, w2, preferred_element_type=jnp.float32)
        y_ref[...] = jnp.maximum(o1 + o2, 0.0)


def _pass_b(lo_ref, hi_ref, x_ref, b_ref, y_ref, o_ref):
    i = pl.program_id(0)
    b = b_ref[0, :, :]
    s_lo = lo_ref[i]
    s_hi = hi_ref[i]

    def body(s, carry):
        ys = y_ref[pl.ds(s, 1), :]
        m = b == s
        o_ref[...] = jnp.where(m, ys, o_ref[...])
        return carry

    jax.lax.fori_loop(s_lo, s_hi + 1, body, 0)
    o_ref[...] = o_ref[...] * x_ref[...]


def kernel(x, batch, W1, W2):
    n, c = x.shape
    nb = n // _BR
    batch3 = batch.reshape(nb, _BR, 1)
    blo = batch3[:, 0, 0]
    bhi = batch3[:, _BR - 1, 0]

    y = pl.pallas_call(
        _pass_a,
        grid=(nb,),
        in_specs=[
            pl.BlockSpec(memory_space=pltpu.SMEM),
            pl.BlockSpec(memory_space=pltpu.SMEM),
            pl.BlockSpec((_BR, c), lambda i: (i, 0)),
            pl.BlockSpec((1, _BR, 1), lambda i: (i, 0, 0)),
            pl.BlockSpec((c, c // 8), lambda i: (0, 0)),
            pl.BlockSpec((c // 8, c), lambda i: (0, 0)),
        ],
        out_specs=pl.BlockSpec((_G, c), lambda i: (0, 0)),
        out_shape=jax.ShapeDtypeStruct((_G, c), jnp.float32),
        scratch_shapes=[
            pltpu.VMEM((_G, c), jnp.float32),
            pltpu.VMEM((_G, c), jnp.float32),
        ],
        compiler_params=pltpu.CompilerParams(
            dimension_semantics=("arbitrary",),
        ),
    )(blo, bhi, x, batch3, W1, W2)

    out = pl.pallas_call(
        _pass_b,
        grid=(nb,),
        in_specs=[
            pl.BlockSpec(memory_space=pltpu.SMEM),
            pl.BlockSpec(memory_space=pltpu.SMEM),
            pl.BlockSpec((_BR, c), lambda i: (i, 0)),
            pl.BlockSpec((1, _BR, 1), lambda i: (i, 0, 0)),
            pl.BlockSpec((_G, c), lambda i: (0, 0)),
        ],
        out_specs=pl.BlockSpec((_BR, c), lambda i: (i, 0)),
        out_shape=jax.ShapeDtypeStruct((n, c), jnp.float32),
        compiler_params=pltpu.CompilerParams(
            dimension_semantics=("arbitrary",),
        ),
    )(blo, bhi, x, batch3, y)
    return out
